# baseline (speedup 1.0000x reference)
import functools

import jax
import jax.numpy as jnp
from jax import lax
from jax.experimental import pallas as pl
from jax.experimental.pallas import tpu as pltpu
from jax.experimental.pallas import tpu_sc as plsc

E = 8
D = 768
F = 2048
T = 2048
P = 2 * T
TM = 256
NT = 24
NPAD = NT * TM
NS = 16
DW = D // 2


def _router_body(x_ref, wr_ref, eidx_ref, ew_ref):
    logits = lax.dot_general(
        x_ref[...], wr_ref[...], (((1,), (1,)), ((), ())),
        preferred_element_type=jnp.float32)
    idx = lax.broadcasted_iota(jnp.int32, logits.shape, 1)
    m1 = jnp.max(logits, axis=1, keepdims=True)
    i1 = jnp.min(jnp.where(logits == m1, idx, E), axis=1, keepdims=True)
    masked = jnp.where(idx == i1, -jnp.inf, logits)
    m2 = jnp.max(masked, axis=1, keepdims=True)
    i2 = jnp.min(jnp.where(masked == m2, idx, E), axis=1, keepdims=True)
    w1 = 1.0 / (1.0 + jnp.exp(m2 - m1))
    eidx_ref[...] = jnp.concatenate([i1, i2], axis=1)
    ew_ref[...] = jnp.concatenate([w1, 1.0 - w1], axis=1)


def _sort_gather_body(ef_hbm, xview_hbm, xs_hbm, pos_hbm, te_hbm,
                      ef_v, z_v, cnt_v, allcnt_v, ptr_v, dest_v, tok_v,
                      te_v, gidx_v, rows_v, counts_sp, sorted_sp, sem):
    cid = lax.axis_index("c")
    sid = lax.axis_index("s")

    @pl.when(cid == 0)
    def _():
        lane = lax.iota(jnp.int32, 16)

        pltpu.sync_copy(ef_hbm.at[pl.ds(sid * 2, 2)], ef_v)
        cnt = jnp.zeros((16,), jnp.int32)
        for j in range(16):
            v = ef_v[j // 8, pl.ds((j % 8) * 16, 16)]
            for e in range(E):
                pc = plsc.cumsum(jnp.where(v == e, 1, 0))[15]
                cnt = cnt + jnp.where(lane == e, pc, 0)
        cnt_v[...] = cnt

        z16 = jnp.zeros((16,), jnp.int32)
        for j in range(24):
            z_v[pl.ds(j * 16, 16)] = z16
        pltpu.sync_copy(z_v, sorted_sp.at[pl.ds(sid * (NPAD // NS), NPAD // NS)])

        pltpu.sync_copy(cnt_v, counts_sp.at[sid])
        plsc.subcore_barrier()

        pltpu.sync_copy(counts_sp, allcnt_v)
        total = jnp.zeros((16,), jnp.int32)
        pref = jnp.zeros((16,), jnp.int32)
        for w in range(NS):
            row = allcnt_v[w, :]
            total = total + row
            pref = pref + jnp.where(w < sid, row, 0)
        padded = ((total + (TM - 1)) // TM) * TM
        incl = plsc.cumsum(padded)
        base = incl - padded
        ptr = base + pref

        for j in range(16):
            v = ef_v[j // 8, pl.ds((j % 8) * 16, 16)]
            ptr_v[...] = ptr
            myp = plsc.load_gather(ptr_v, [v])
            dest = jnp.zeros((16,), jnp.int32)
            for e in range(E):
                m = v == e
                r = plsc.cumsum(jnp.where(m, 1, 0))
                dest = jnp.where(m, myp + r - 1, dest)
                ptr = ptr + jnp.where(lane == e, r[15], 0)
            dest_v[j // 8, pl.ds((j % 8) * 16, 16)] = dest
            tok = (sid * 256 + j * 16 + lane) // 2
            tok_v[j // 8, pl.ds((j % 8) * 16, 16)] = tok

        pltpu.sync_copy(dest_v, pos_hbm.at[pl.ds(sid * 2, 2)])
        for r in range(2):
            pltpu.sync_copy(tok_v.at[r], sorted_sp.at[dest_v.at[r]])

        @pl.when(sid == 0)
        def _te():
            for h in range(2):
                row0 = (lane + h * 16) * TM
                te = jnp.full((16,), E - 1, jnp.int32)
                for e in range(E):
                    be = jnp.sum(jnp.where(lane == e, base, 0))
                    pe = jnp.sum(jnp.where(lane == e, padded, 0))
                    m = (row0 >= be) & (row0 < be + pe)
                    te = jnp.where(m, e, te)
                te_v[pl.ds(h * 16, 16)] = te
            pltpu.sync_copy(te_v, te_hbm)

        plsc.subcore_barrier()

        for ch in range(3):
            start = sid * (NPAD // NS) + ch * 128
            pltpu.sync_copy(sorted_sp.at[pl.ds(start, 128)], gidx_v)
            pltpu.async_copy(xview_hbm.at[gidx_v], rows_v, sem).wait()
            pltpu.sync_copy(rows_v, xs_hbm.at[pl.ds(start, 128)])


def _expert_body(te_ref, xs_ref, gu_ref, dn_ref, yw_ref):
    del te_ref
    xb = xs_ref[...]
    h = jnp.dot(xb, gu_ref[0], preferred_element_type=jnp.float32)
    gate = h[:, :F]
    up = h[:, F:]
    act = (gate * jax.nn.sigmoid(gate) * up).astype(jnp.bfloat16)
    yw_ref[...] = jnp.dot(act, dn_ref[0], preferred_element_type=jnp.float32)


def _combine_body(yw_hbm, pos_hbm, ew_hbm, out_hbm,
                  posv, eww, rows_v, obuf, sem):
    cid = lax.axis_index("c")
    sid = lax.axis_index("s")
    wid = sid * 2 + cid
    for ch in range(2):
        tok0 = wid * 64 + ch * 32
        pltpu.sync_copy(pos_hbm.at[wid, pl.ds(ch * 64, 64)], posv)
        pltpu.sync_copy(ew_hbm.at[wid, pl.ds(ch * 64, 64)], eww.at[pl.ds(0, 64)])
        pltpu.async_copy(yw_hbm.at[posv], rows_v, sem).wait()

        def body(i, carry):
            wv = eww[pl.ds(2 * i, 16)]
            wa = wv[0]
            wb = wv[1]
            for q in range(D // 16):
                a = rows_v[2 * i, pl.ds(q * 16, 16)]
                b = rows_v[2 * i + 1, pl.ds(q * 16, 16)]
                obuf[i, pl.ds(q * 16, 16)] = a * wa + b * wb
            return carry

        lax.fori_loop(0, 32, body, 0)
        pltpu.sync_copy(obuf, out_hbm.at[pl.ds(tok0, 32)])


def kernel(hidden_states, router_weight, gate_up_proj, down_proj):
    B, S, _ = hidden_states.shape
    x32 = hidden_states.reshape(B * S, D)
    xb = x32.astype(jnp.bfloat16)
    gub = gate_up_proj.astype(jnp.bfloat16)
    dnb = down_proj.astype(jnp.bfloat16)
    xview = lax.bitcast_convert_type(xb.reshape(T, DW, 2), jnp.float32)

    eidx, ew = pl.pallas_call(
        _router_body,
        in_specs=[pl.BlockSpec((T, D), lambda: (0, 0)),
                  pl.BlockSpec((E, D), lambda: (0, 0))],
        out_specs=[pl.BlockSpec((T, 2), lambda: (0, 0)),
                   pl.BlockSpec((T, 2), lambda: (0, 0))],
        out_shape=[jax.ShapeDtypeStruct((T, 2), jnp.int32),
                   jax.ShapeDtypeStruct((T, 2), jnp.float32)],
    )(x32, router_weight)

    ef = eidx.reshape(32, 128)
    ew2 = ew.reshape(32, 128)

    mesh = plsc.VectorSubcoreMesh(core_axis_name="c", subcore_axis_name="s",
                                  num_cores=2, num_subcores=NS)
    sort_gather = functools.partial(
        pl.kernel,
        out_type=[jax.ShapeDtypeStruct((NPAD, DW), jnp.float32),
                  jax.ShapeDtypeStruct((32, 128), jnp.int32),
                  jax.ShapeDtypeStruct((32,), jnp.int32)],
        mesh=mesh,
        scratch_types=[
            pltpu.VMEM((2, 128), jnp.int32),
            pltpu.VMEM((NPAD // NS,), jnp.int32),
            pltpu.VMEM((16,), jnp.int32),
            pltpu.VMEM((16, 16), jnp.int32),
            pltpu.VMEM((16,), jnp.int32),
            pltpu.VMEM((2, 128), jnp.int32),
            pltpu.VMEM((2, 128), jnp.int32),
            pltpu.VMEM((32,), jnp.int32),
            pltpu.VMEM((128,), jnp.int32),
            pltpu.VMEM((128, DW), jnp.float32),
            pltpu.VMEM_SHARED((NS, 16), jnp.int32),
            pltpu.VMEM_SHARED((NPAD,), jnp.int32),
            pltpu.SemaphoreType.DMA,
        ],
        compiler_params=pltpu.CompilerParams(needs_layout_passes=False),
    )(_sort_gather_body)
    xs_view, pos, te = sort_gather(ef, xview)

    xs_b = lax.bitcast_convert_type(xs_view, jnp.bfloat16).reshape(NPAD, D)

    grid_spec = pltpu.PrefetchScalarGridSpec(
        num_scalar_prefetch=1,
        grid=(NT,),
        in_specs=[
            pl.BlockSpec((TM, D), lambda j, te_r: (j, 0)),
            pl.BlockSpec((1, D, 2 * F), lambda j, te_r: (te_r[j], 0, 0)),
            pl.BlockSpec((1, F, D), lambda j, te_r: (te_r[j], 0, 0)),
        ],
        out_specs=pl.BlockSpec((TM, D), lambda j, te_r: (j, 0)),
    )
    yw = pl.pallas_call(
        _expert_body,
        grid_spec=grid_spec,
        out_shape=jax.ShapeDtypeStruct((NPAD, D), jnp.float32),
        compiler_params=pltpu.CompilerParams(
            dimension_semantics=("arbitrary",)),
    )(te, xs_b, gub, dnb)

    combine = functools.partial(
        pl.kernel,
        out_type=jax.ShapeDtypeStruct((T, D), jnp.float32),
        mesh=mesh,
        scratch_types=[
            pltpu.VMEM((64,), jnp.int32),
            pltpu.VMEM((80,), jnp.float32),
            pltpu.VMEM((64, D), jnp.float32),
            pltpu.VMEM((32, D), jnp.float32),
            pltpu.SemaphoreType.DMA,
        ],
        compiler_params=pltpu.CompilerParams(needs_layout_passes=False),
    )(_combine_body)
    out = combine(yw, pos, ew2)
    return out.reshape(B, S, D)


# speedup vs baseline: 2.2306x; 2.2306x over previous
import functools

import jax
import jax.numpy as jnp
from jax import lax
from jax.experimental import pallas as pl
from jax.experimental.pallas import tpu as pltpu
from jax.experimental.pallas import tpu_sc as plsc

E = 8
D = 768
F = 2048
T = 2048
P = 2 * T
TM = 256
NT = 24
NPAD = NT * TM
NS = 16


def _router_body(x_ref, wr_ref, e1_ref, e2_ref, w1_ref, w2_ref, cnt_ref):
    logits = lax.dot_general(
        x_ref[...], wr_ref[...], (((1,), (1,)), ((), ())),
        preferred_element_type=jnp.float32)
    idx = lax.broadcasted_iota(jnp.int32, logits.shape, 1)
    m1 = jnp.max(logits, axis=1, keepdims=True)
    i1 = jnp.min(jnp.where(logits == m1, idx, E), axis=1, keepdims=True)
    masked = jnp.where(idx == i1, -jnp.inf, logits)
    m2 = jnp.max(masked, axis=1, keepdims=True)
    i2 = jnp.min(jnp.where(masked == m2, idx, E), axis=1, keepdims=True)
    w1 = 1.0 / (1.0 + jnp.exp(m2 - m1))
    e1_ref[...] = i1
    e2_ref[...] = i2
    w1_ref[...] = w1
    w2_ref[...] = 1.0 - w1
    tb = lax.broadcasted_iota(jnp.int32, (T, NS), 1)
    tokb = lax.broadcasted_iota(jnp.int32, (T, NS), 0) // 256
    bmask = (tb == tokb).astype(jnp.float32)
    m1f = (idx == i1).astype(jnp.float32)
    m2f = (idx == i2).astype(jnp.float32)
    h1 = lax.dot_general(bmask, m1f, (((0,), (0,)), ((), ())),
                         preferred_element_type=jnp.float32)
    h2 = lax.dot_general(bmask, m2f, (((0,), (0,)), ((), ())),
                         preferred_element_type=jnp.float32)
    h1p = jnp.pad(h1[:8], ((0, 0), (0, 16 - E)))
    h2p = jnp.pad(h2[:8], ((0, 0), (0, 16 - E)))
    cnt_ref[...] = jnp.concatenate([h1p, h2p], axis=0).astype(jnp.int32)


def _sort_body(ef_hbm, cnt_hbm, pos_hbm, te_hbm,
               ef_v, allcnt_v, ptr_v, dest_v, te_v, sem):
    del sem
    cid = lax.axis_index("c")
    sid = lax.axis_index("s")

    @pl.when(cid == 0)
    def _():
        lane = lax.iota(jnp.int32, 16)

        pltpu.sync_copy(ef_hbm.at[pl.ds(sid * 2, 2)], ef_v)
        pltpu.sync_copy(cnt_hbm, allcnt_v)
        total = jnp.zeros((16,), jnp.int32)
        pref = jnp.zeros((16,), jnp.int32)
        for w in range(NS):
            row = allcnt_v[w, :]
            total = total + row
            pref = pref + jnp.where(w < sid, row, 0)
        padded = ((total + (TM - 1)) // TM) * TM
        incl = plsc.cumsum(padded)
        base = incl - padded
        ptr = base + pref

        for j in range(16):
            v = ef_v[j // 8, pl.ds((j % 8) * 16, 16)]
            ptr_v[...] = ptr
            myp = plsc.load_gather(ptr_v, [v])
            dest = jnp.zeros((16,), jnp.int32)
            for e in range(E):
                m = v == e
                r = plsc.cumsum(jnp.where(m, 1, 0))
                dest = jnp.where(m, myp + r - 1, dest)
                ptr = ptr + jnp.where(lane == e, r[15], 0)
            dest_v[j // 8, pl.ds((j % 8) * 16, 16)] = dest

        for r in range(2):
            pltpu.sync_copy(dest_v.at[r],
                            pos_hbm.at[pl.ds(sid * 256 + r * 128, 128)])

        @pl.when(sid == 0)
        def _te():
            for h in range(2):
                row0 = (lane + h * 16) * TM
                te = jnp.full((16,), 2 * E - 1, jnp.int32)
                for e in range(E):
                    be = jnp.sum(jnp.where(lane == e, base, 0))
                    pe = jnp.sum(jnp.where(lane == e, padded, 0))
                    m = (row0 >= be) & (row0 < be + pe)
                    te = jnp.where(m, e, te)
                te_v[pl.ds(h * 16, 16)] = te
            pltpu.sync_copy(te_v, te_hbm)


def _scatter_body(pos_hbm, x_hbm, xs_hbm, p_v, rows_v, sem):
    cid = lax.axis_index("c")
    sid = lax.axis_index("s")
    wid = sid * 2 + cid
    t0 = (wid & (NS - 1)) * 128
    pltpu.sync_copy(pos_hbm.at[pl.ds(wid * 128, 128)], p_v)
    pltpu.sync_copy(x_hbm.at[pl.ds(t0, 128)], rows_v)
    pltpu.async_copy(rows_v, xs_hbm.at[p_v], sem).wait()


def _expert_body(te_ref, xs_ref, gu_ref, dn_ref, yw_ref):
    j = pl.program_id(0)

    @pl.when(te_ref[j] < E)
    def _():
        xb = xs_ref[...].astype(jnp.bfloat16)
        h = jnp.dot(xb, gu_ref[0], preferred_element_type=jnp.float32)
        gate = h[:, :F]
        up = h[:, F:]
        act = (gate * jax.nn.sigmoid(gate) * up).astype(jnp.bfloat16)
        yw_ref[...] = jnp.dot(act, dn_ref[0], preferred_element_type=jnp.float32)


def _combine_body(yw_hbm, pos_hbm, w1_hbm, w2_hbm, out_hbm,
                  p0_v, p1_v, wa_v, wb_v, b0, b1, sem0, sem1):
    cid = lax.axis_index("c")
    sid = lax.axis_index("s")
    wid = sid * 2 + cid
    pltpu.sync_copy(pos_hbm.at[pl.ds(wid * 64, 64)], p0_v)
    pltpu.sync_copy(pos_hbm.at[pl.ds(T + wid * 64, 64)], p1_v)
    pltpu.sync_copy(w1_hbm.at[pl.ds(wid * 64, 64)], wa_v.at[pl.ds(0, 64)])
    pltpu.sync_copy(w2_hbm.at[pl.ds(wid * 64, 64)], wb_v.at[pl.ds(0, 64)])
    d0 = pltpu.async_copy(yw_hbm.at[p0_v], b0, sem0)
    d1 = pltpu.async_copy(yw_hbm.at[p1_v], b1, sem1)
    d0.wait()
    d1.wait()

    def body(i, carry):
        wa = wa_v[pl.ds(i, 16)][0]
        wb = wb_v[pl.ds(i, 16)][0]
        for q in range(D // 16):
            b0[i, pl.ds(q * 16, 16)] = (b0[i, pl.ds(q * 16, 16)] * wa
                                        + b1[i, pl.ds(q * 16, 16)] * wb)
        return carry

    lax.fori_loop(0, 64, body, 0)
    pltpu.sync_copy(b0, out_hbm.at[pl.ds(wid * 64, 64)])


def kernel(hidden_states, router_weight, gate_up_proj, down_proj):
    B, S, _ = hidden_states.shape
    x32 = hidden_states.reshape(B * S, D)
    gub = gate_up_proj.astype(jnp.bfloat16)
    dnb = down_proj.astype(jnp.bfloat16)

    e1, e2, w1, w2, cnt16 = pl.pallas_call(
        _router_body,
        in_specs=[pl.BlockSpec((T, D), lambda: (0, 0)),
                  pl.BlockSpec((E, D), lambda: (0, 0))],
        out_specs=[pl.BlockSpec((T, 1), lambda: (0, 0))] * 4
        + [pl.BlockSpec((NS, 16), lambda: (0, 0))],
        out_shape=[jax.ShapeDtypeStruct((T, 1), jnp.int32),
                   jax.ShapeDtypeStruct((T, 1), jnp.int32),
                   jax.ShapeDtypeStruct((T, 1), jnp.float32),
                   jax.ShapeDtypeStruct((T, 1), jnp.float32),
                   jax.ShapeDtypeStruct((NS, 16), jnp.int32)],
    )(x32, router_weight)

    ef = jnp.concatenate([e1, e2], axis=0).reshape(32, 128)

    mesh = plsc.VectorSubcoreMesh(core_axis_name="c", subcore_axis_name="s",
                                  num_cores=2, num_subcores=NS)
    sc_params = pltpu.CompilerParams(needs_layout_passes=False)

    sort = functools.partial(
        pl.kernel,
        out_type=[jax.ShapeDtypeStruct((P,), jnp.int32),
                  jax.ShapeDtypeStruct((32,), jnp.int32)],
        mesh=mesh,
        scratch_types=[
            pltpu.VMEM((2, 128), jnp.int32),
            pltpu.VMEM((16, 16), jnp.int32),
            pltpu.VMEM((16,), jnp.int32),
            pltpu.VMEM((2, 128), jnp.int32),
            pltpu.VMEM((32,), jnp.int32),
            pltpu.SemaphoreType.DMA,
        ],
        compiler_params=sc_params,
    )(_sort_body)
    pos, te = sort(ef, cnt16)

    scatter = functools.partial(
        pl.kernel,
        out_type=jax.ShapeDtypeStruct((NPAD, D), jnp.float32),
        mesh=mesh,
        scratch_types=[
            pltpu.VMEM((128,), jnp.int32),
            pltpu.VMEM((128, D), jnp.float32),
            pltpu.SemaphoreType.DMA,
        ],
        compiler_params=sc_params,
    )(_scatter_body)
    xs = scatter(pos, x32)

    grid_spec = pltpu.PrefetchScalarGridSpec(
        num_scalar_prefetch=1,
        grid=(NT,),
        in_specs=[
            pl.BlockSpec((TM, D), lambda j, te_r: (j, 0)),
            pl.BlockSpec((1, D, 2 * F), lambda j, te_r: (te_r[j] & 7, 0, 0)),
            pl.BlockSpec((1, F, D), lambda j, te_r: (te_r[j] & 7, 0, 0)),
        ],
        out_specs=pl.BlockSpec((TM, D), lambda j, te_r: (j, 0)),
    )
    yw = pl.pallas_call(
        _expert_body,
        grid_spec=grid_spec,
        out_shape=jax.ShapeDtypeStruct((NPAD, D), jnp.float32),
        compiler_params=pltpu.CompilerParams(
            dimension_semantics=("arbitrary",)),
    )(te, xs, gub, dnb)

    combine = functools.partial(
        pl.kernel,
        out_type=jax.ShapeDtypeStruct((T, D), jnp.float32),
        mesh=mesh,
        scratch_types=[
            pltpu.VMEM((64,), jnp.int32),
            pltpu.VMEM((64,), jnp.int32),
            pltpu.VMEM((80,), jnp.float32),
            pltpu.VMEM((80,), jnp.float32),
            pltpu.VMEM((64, D), jnp.float32),
            pltpu.VMEM((64, D), jnp.float32),
            pltpu.SemaphoreType.DMA,
            pltpu.SemaphoreType.DMA,
        ],
        compiler_params=sc_params,
    )(_combine_body)
    out = combine(yw, pos, w1.reshape(T), w2.reshape(T))
    return out.reshape(B, S, D)


# speedup vs baseline: 2.7763x; 1.2447x over previous
import functools

import jax
import jax.numpy as jnp
from jax import lax
from jax.experimental import pallas as pl
from jax.experimental.pallas import tpu as pltpu
from jax.experimental.pallas import tpu_sc as plsc

E = 8
D = 768
F = 2048
T = 2048
P = 2 * T
TM = 256
NT = 24
NPAD = NT * TM
NS = 16


def _router_body(x_ref, wr_ref, e1_ref, e2_ref, w1_ref, w2_ref, cnt_ref):
    logits = lax.dot_general(
        x_ref[...], wr_ref[...], (((1,), (1,)), ((), ())),
        preferred_element_type=jnp.float32)
    idx = lax.broadcasted_iota(jnp.int32, logits.shape, 1)
    m1 = jnp.max(logits, axis=1, keepdims=True)
    i1 = jnp.min(jnp.where(logits == m1, idx, E), axis=1, keepdims=True)
    masked = jnp.where(idx == i1, -jnp.inf, logits)
    m2 = jnp.max(masked, axis=1, keepdims=True)
    i2 = jnp.min(jnp.where(masked == m2, idx, E), axis=1, keepdims=True)
    w1 = 1.0 / (1.0 + jnp.exp(m2 - m1))
    e1_ref[...] = i1
    e2_ref[...] = i2
    w1_ref[...] = w1
    w2_ref[...] = 1.0 - w1
    tb = lax.broadcasted_iota(jnp.int32, (T, NS), 1)
    tokb = lax.broadcasted_iota(jnp.int32, (T, NS), 0) // 128
    bmask = (tb == tokb).astype(jnp.float32)
    m1f = (idx == i1).astype(jnp.float32)
    m2f = (idx == i2).astype(jnp.float32)
    h1 = lax.dot_general(bmask, m1f, (((0,), (0,)), ((), ())),
                         preferred_element_type=jnp.float32)
    h2 = lax.dot_general(bmask, m2f, (((0,), (0,)), ((), ())),
                         preferred_element_type=jnp.float32)
    h1p = jnp.pad(h1, ((0, 0), (0, 16 - E)))
    h2p = jnp.pad(h2, ((0, 0), (0, 16 - E)))
    cnt_ref[...] = jnp.concatenate([h1p, h2p], axis=0).astype(jnp.int32)


def _sort_scatter_body(ef_hbm, cnt_hbm, x_hbm, xs_hbm, pos_hbm, te_hbm,
                       ef_v, allcnt_v, ptr_v, dest_v, te_v, rows_v, sem):
    cid = lax.axis_index("c")
    sid = lax.axis_index("s")
    wid = sid * 2 + cid
    lane = lax.iota(jnp.int32, 16)

    t0 = (wid & (NS - 1)) * 128
    drows = pltpu.async_copy(x_hbm.at[pl.ds(t0, 128)], rows_v, sem)
    pltpu.sync_copy(ef_hbm.at[wid], ef_v)
    pltpu.sync_copy(cnt_hbm, allcnt_v)
    total = jnp.zeros((16,), jnp.int32)
    pref = jnp.zeros((16,), jnp.int32)
    for w in range(2 * NS):
        row = allcnt_v[w, :]
        total = total + row
        pref = pref + jnp.where(w < wid, row, 0)
    padded = ((total + (TM - 1)) // TM) * TM
    incl = plsc.cumsum(padded)
    base = incl - padded
    ptr = base + pref

    for j in range(8):
        v = ef_v[pl.ds(j * 16, 16)]
        ptr_v[...] = ptr
        myp = plsc.load_gather(ptr_v, [v])
        dest = jnp.zeros((16,), jnp.int32)
        for e in range(E):
            m = v == e
            r = plsc.cumsum(jnp.where(m, 1, 0))
            dest = jnp.where(m, myp + r - 1, dest)
            ptr = ptr + jnp.where(lane == e, r[15], 0)
        dest_v[0, pl.ds(j * 16, 16)] = dest

    pltpu.sync_copy(dest_v.at[0], pos_hbm.at[pl.ds(wid * 128, 128)])
    drows.wait()
    pltpu.async_copy(rows_v, xs_hbm.at[dest_v.at[0]], sem).wait()

    @pl.when((cid == 0) & (sid == 0))
    def _te():
        for h in range(2):
            row0 = (lane + h * 16) * TM
            te = jnp.full((16,), 2 * E - 1, jnp.int32)
            for e in range(E):
                be = jnp.sum(jnp.where(lane == e, base, 0))
                pe = jnp.sum(jnp.where(lane == e, padded, 0))
                m = (row0 >= be) & (row0 < be + pe)
                te = jnp.where(m, e, te)
            te_v[pl.ds(h * 16, 16)] = te
        pltpu.sync_copy(te_v, te_hbm)


def _expert_body(te_ref, xs_ref, gu_ref, dn_ref, yw_ref, gub_s, dnb_s):
    j = pl.program_id(0)
    te = te_ref[j]

    @pl.when(te < E)
    def _():
        changed = jnp.logical_or(j == 0, te_ref[jnp.maximum(j - 1, 0)] != te)

        @pl.when(changed)
        def _cvt():
            gub_s[...] = gu_ref[0].astype(jnp.bfloat16)
            dnb_s[...] = dn_ref[0].astype(jnp.bfloat16)

        xb = xs_ref[...].astype(jnp.bfloat16)
        FC = F // 2
        acc = None
        for c in range(2):
            g = jnp.dot(xb, gub_s[:, c * FC:(c + 1) * FC],
                        preferred_element_type=jnp.float32)
            u = jnp.dot(xb, gub_s[:, F + c * FC:F + (c + 1) * FC],
                        preferred_element_type=jnp.float32)
            a = (g * jax.nn.sigmoid(g) * u).astype(jnp.bfloat16)
            yc = jnp.dot(a, dnb_s[c * FC:(c + 1) * FC, :],
                         preferred_element_type=jnp.float32)
            acc = yc if acc is None else acc + yc
        yw_ref[...] = acc


def _combine_body(yw_hbm, pos_hbm, w1_hbm, w2_hbm, out_hbm,
                  p0_v, p1_v, wa_v, wb_v, b0, b1, sem0, sem1):
    cid = lax.axis_index("c")
    sid = lax.axis_index("s")
    wid = sid * 2 + cid
    pltpu.sync_copy(pos_hbm.at[pl.ds(wid * 64, 64)], p0_v)
    pltpu.sync_copy(pos_hbm.at[pl.ds(T + wid * 64, 64)], p1_v)
    pltpu.sync_copy(w1_hbm.at[pl.ds(wid * 64, 64)], wa_v.at[pl.ds(0, 64)])
    pltpu.sync_copy(w2_hbm.at[pl.ds(wid * 64, 64)], wb_v.at[pl.ds(0, 64)])
    d0 = pltpu.async_copy(yw_hbm.at[p0_v], b0, sem0)
    d1 = pltpu.async_copy(yw_hbm.at[p1_v], b1, sem1)
    d0.wait()
    d1.wait()

    def body(i, carry):
        wa = wa_v[pl.ds(i, 16)][0]
        wb = wb_v[pl.ds(i, 16)][0]
        for q in range(D // 16):
            b0[i, pl.ds(q * 16, 16)] = (b0[i, pl.ds(q * 16, 16)] * wa
                                        + b1[i, pl.ds(q * 16, 16)] * wb)
        return carry

    lax.fori_loop(0, 64, body, 0)
    pltpu.sync_copy(b0, out_hbm.at[pl.ds(wid * 64, 64)])


def kernel(hidden_states, router_weight, gate_up_proj, down_proj):
    B, S, _ = hidden_states.shape
    x32 = hidden_states.reshape(B * S, D)

    e1, e2, w1, w2, cnt16 = pl.pallas_call(
        _router_body,
        in_specs=[pl.BlockSpec((T, D), lambda: (0, 0)),
                  pl.BlockSpec((E, D), lambda: (0, 0))],
        out_specs=[pl.BlockSpec((T, 1), lambda: (0, 0))] * 4
        + [pl.BlockSpec((2 * NS, 16), lambda: (0, 0))],
        out_shape=[jax.ShapeDtypeStruct((T, 1), jnp.int32),
                   jax.ShapeDtypeStruct((T, 1), jnp.int32),
                   jax.ShapeDtypeStruct((T, 1), jnp.float32),
                   jax.ShapeDtypeStruct((T, 1), jnp.float32),
                   jax.ShapeDtypeStruct((2 * NS, 16), jnp.int32)],
    )(x32, router_weight)

    ef = jnp.concatenate([e1, e2], axis=0).reshape(32, 128)

    mesh = plsc.VectorSubcoreMesh(core_axis_name="c", subcore_axis_name="s",
                                  num_cores=2, num_subcores=NS)
    sc_params = pltpu.CompilerParams(needs_layout_passes=False)

    sort_scatter = functools.partial(
        pl.kernel,
        out_type=[jax.ShapeDtypeStruct((NPAD, D), jnp.float32),
                  jax.ShapeDtypeStruct((P,), jnp.int32),
                  jax.ShapeDtypeStruct((32,), jnp.int32)],
        mesh=mesh,
        scratch_types=[
            pltpu.VMEM((128,), jnp.int32),
            pltpu.VMEM((2 * NS, 16), jnp.int32),
            pltpu.VMEM((16,), jnp.int32),
            pltpu.VMEM((2, 128), jnp.int32),
            pltpu.VMEM((32,), jnp.int32),
            pltpu.VMEM((128, D), jnp.float32),
            pltpu.SemaphoreType.DMA,
        ],
        compiler_params=sc_params,
    )(_sort_scatter_body)
    xs, pos, te = sort_scatter(ef, cnt16, x32)

    grid_spec = pltpu.PrefetchScalarGridSpec(
        num_scalar_prefetch=1,
        grid=(NT,),
        in_specs=[
            pl.BlockSpec((TM, D), lambda j, te_r: (j, 0)),
            pl.BlockSpec((1, D, 2 * F), lambda j, te_r: (te_r[j] & 7, 0, 0)),
            pl.BlockSpec((1, F, D), lambda j, te_r: (te_r[j] & 7, 0, 0)),
        ],
        out_specs=pl.BlockSpec((TM, D), lambda j, te_r: (j, 0)),
        scratch_shapes=[pltpu.VMEM((D, 2 * F), jnp.bfloat16),
                        pltpu.VMEM((F, D), jnp.bfloat16)],
    )
    yw = pl.pallas_call(
        _expert_body,
        grid_spec=grid_spec,
        out_shape=jax.ShapeDtypeStruct((NPAD, D), jnp.float32),
        compiler_params=pltpu.CompilerParams(
            dimension_semantics=("arbitrary",)),
    )(te, xs, gate_up_proj, down_proj)

    combine = functools.partial(
        pl.kernel,
        out_type=jax.ShapeDtypeStruct((T, D), jnp.float32),
        mesh=mesh,
        scratch_types=[
            pltpu.VMEM((64,), jnp.int32),
            pltpu.VMEM((64,), jnp.int32),
            pltpu.VMEM((80,), jnp.float32),
            pltpu.VMEM((80,), jnp.float32),
            pltpu.VMEM((64, D), jnp.float32),
            pltpu.VMEM((64, D), jnp.float32),
            pltpu.SemaphoreType.DMA,
            pltpu.SemaphoreType.DMA,
        ],
        compiler_params=sc_params,
    )(_combine_body)
    out = combine(yw, pos, w1.reshape(T), w2.reshape(T))
    return out.reshape(B, S, D)
